# baseline (speedup 1.0000x reference)
import functools

import jax
import jax.numpy as jnp
from jax import lax
from jax.experimental import pallas as pl
from jax.experimental.pallas import tpu as pltpu
from jax.experimental.pallas import tpu_sc as plsc

VOCAB = 1000000
LATENT = 32
BATCH = 16384

NC = 2
NS = 16
NW = NC * NS
L = 16

ROWS = BATCH // NW
VPW = VOCAB // NW
PW = 512
N_PIECES = (VPW + 256 + PW - 1) // PW
MCAP = 4096
NB = 62
BCAP = 64
TRASH = BATCH
SROWS = BATCH + 8
BIGV = 1 << 30
TAIL_LO = (VOCAB // 128) * 128
TAIL_W = VOCAB - TAIL_LO
LAST_PIECE = ((VOCAB - TAIL_W - PW) // 128) * 128


def _phase_a(ids_t, table_t, tail_t):
    mesh = plsc.VectorSubcoreMesh(core_axis_name="c", subcore_axis_name="s")

    @functools.partial(
        pl.kernel,
        mesh=mesh,
        compiler_params=pltpu.CompilerParams(
            use_tc_tiling_on_sc=True, needs_layout_passes=False),
        out_type=(
            jax.ShapeDtypeStruct((SROWS, 128), jnp.float32),
            jax.ShapeDtypeStruct((SROWS, 128), jnp.float32),
        ),
        scratch_types=[
            pltpu.VMEM((2, BATCH // 8), jnp.int32),
            pltpu.VMEM((NB * BCAP,), jnp.int32),
            pltpu.VMEM((NB * BCAP,), jnp.int32),
            pltpu.VMEM((NB * BCAP // L, L), jnp.int32),
            pltpu.VMEM((NB * BCAP // L, L), jnp.int32),
            pltpu.VMEM((64,), jnp.int32),
            pltpu.VMEM((64,), jnp.int32),
            pltpu.VMEM((L,), jnp.int32),
            pltpu.VMEM((LATENT, PW), jnp.float32),
            pltpu.VMEM((LATENT, PW), jnp.float32),
            pltpu.VMEM((BCAP, 128), jnp.float32),
            pltpu.VMEM((BCAP, 128), jnp.float32),
            pltpu.VMEM((LATENT, TAIL_W), jnp.float32),
            pltpu.SemaphoreType.DMA,
            pltpu.SemaphoreType.DMA,
            pltpu.SemaphoreType.DMA,
        ],
    )
    def k(ids_hbm, table_hbm, tail_hbm, s1, s2,
          ids_v, bv1, bv2, bp1, bp2, bc1, bc2, stage_v,
          piece_a, piece_b, e1a, e2a, tail_v, sem_a, sem_b, sem_s):
        wid = lax.axis_index("s") * NC + lax.axis_index("c")
        lo = wid * VPW
        hi = lo + VPW
        win_lo = (lo // 128) * 128
        lane = lax.iota(jnp.int32, L)

        def read_lo(p):
            return pl.multiple_of(
                jnp.minimum(win_lo + p * PW, LAST_PIECE), 128)

        def start_piece(p, buf, sem):
            return pltpu.make_async_copy(
                table_hbm.at[:, pl.ds(read_lo(p), PW)], buf, sem).start()

        def wait_piece(buf, sem):
            pltpu.make_async_copy(
                table_hbm.at[:, pl.ds(0, PW)], buf, sem).wait()

        start_piece(0, piece_a, sem_a)
        start_piece(1, piece_b, sem_b)

        zeros = lane * 0
        trash = lane * 0 + TRASH
        for q in range(64 // L):
            bc1[pl.ds(q * L, L)] = zeros
            bc2[pl.ds(q * L, L)] = zeros

        def fill_body(q, _):
            bp1[q, pl.ds(0, L)] = trash
            bp2[q, pl.ds(0, L)] = trash
            return 0

        lax.fori_loop(0, NB * BCAP // L, fill_body, 0)

        def route(v, pos_base, bv, bp, bc):
            m = (v >= lo) & (v < hi)
            mc = jnp.sum(m.astype(jnp.int32), axis=0)

            @pl.when(mc > 0)
            def _():
                stage_v[pl.ds(0, L)] = v

                def drain(_, m32):
                    i = plsc.all_reduce_ffs(m32 > 0)
                    vk = plsc.load_gather(stage_v, [i])
                    pk = i + pos_base
                    b = lax.shift_right_logical(vk - win_lo, 9)
                    cnt = plsc.load_gather(bc, [b])
                    addr = b * BCAP + cnt
                    lane0 = lane < 1
                    plsc.store_scatter(bv, [addr], vk, mask=lane0)
                    plsc.store_scatter(
                        bp,
                        [lax.shift_right_logical(addr, 4), addr & (L - 1)],
                        pk, mask=lane0)
                    plsc.store_scatter(bc, [b], cnt + 1, mask=lane0)
                    return m32 & jnp.where(lane == i, 0, 1)

                lax.fori_loop(0, mc, drain, m.astype(jnp.int32))

        for ch in range(8):
            pltpu.sync_copy(
                ids_hbm.at[:, pl.ds(ch * (BATCH // 8), BATCH // 8)], ids_v)

            def scan_body(kk, _, _base=ch * (BATCH // 8)):
                route(ids_v[0, pl.ds(kk * L, L)], _base + kk * L,
                      bv1, bp1, bc1)
                route(ids_v[1, pl.ds(kk * L, L)], _base + kk * L,
                      bv2, bp2, bc2)
                return 0

            lax.fori_loop(0, BATCH // 8 // L, scan_body, 0)

        def extract(p, pref, piece_lo, bv, bp, bc, sdst, ebuf, inline_wait):
            nb = jnp.max(plsc.load_gather(bc, [zeros + p]), axis=0)

            def ext_body(kk, _):
                vk = plsc.load_gather(bv, [zeros + p * BCAP + kk])
                col = vk - piece_lo
                ebuf[kk, pl.ds(0, L)] = plsc.load_gather(pref, [lane, col])
                ebuf[kk, pl.ds(L, L)] = plsc.load_gather(pref, [lane + L, col])
                return 0

            lax.fori_loop(0, nb, ext_body, 0)

            def sc_body(cc, _):
                cp = pltpu.make_async_copy(
                    ebuf.at[pl.ds(cc * L, L)],
                    sdst.at[bp.at[p * (BCAP // L) + cc]], sem_s)
                cp.start()
                if inline_wait:
                    cp.wait()
                return 0

            lax.fori_loop(0, (nb + L - 1) // L, sc_body, 0)

        def extract_both(p, pref, ebuf1, ebuf2):
            @pl.when(win_lo + p * PW + PW <= TAIL_LO)
            def _():
                piece_lo = win_lo + p * PW
                extract(p, pref, piece_lo, bv1, bp1, bc1, s1, ebuf1, False)
                extract(p, pref, piece_lo, bv2, bp2, bc2, s2, ebuf2, False)

        def n_fired(p):
            ok = win_lo + p * PW + PW <= TAIL_LO
            n1 = jnp.max(plsc.load_gather(bc1, [zeros + p]), axis=0)
            n2 = jnp.max(plsc.load_gather(bc2, [zeros + p]), axis=0)
            n = (n1 + L - 1) // L + (n2 + L - 1) // L
            return jnp.where(ok, n, 0)

        def drain(n):
            def wait_body(_, __):
                pltpu.make_async_copy(
                    e1a.at[pl.ds(0, L)], s1.at[bp1.at[0]], sem_s).wait()
                return 0

            lax.fori_loop(0, n, wait_body, 0)

        def pair_body(pq, _):
            p0 = 2 * pq
            p1 = p0 + 1
            wait_piece(piece_a, sem_a)
            extract_both(p0, piece_a, e1a, e2a)
            drain(n_fired(p0))

            @pl.when(p0 + 2 <= N_PIECES - 2)
            def _():
                start_piece(p0 + 2, piece_a, sem_a)

            wait_piece(piece_b, sem_b)
            extract_both(p1, piece_b, e1a, e2a)
            drain(n_fired(p1))

            @pl.when(p1 + 2 <= N_PIECES - 1)
            def _():
                start_piece(p1 + 2, piece_b, sem_b)

            return 0

        lax.fori_loop(0, N_PIECES // 2, pair_body, 0)

        pltpu.sync_copy(tail_hbm, tail_v)

        @pl.when(win_lo + (N_PIECES - 1) * PW + PW > TAIL_LO)
        def _():
            extract(N_PIECES - 1, tail_v, TAIL_LO, bv1, bp1, bc1, s1, e1a, True)
            extract(N_PIECES - 1, tail_v, TAIL_LO, bv2, bp2, bc2, s2, e2a, True)

    return k(ids_t, table_t, tail_t)


def _phase_b(s1, s2):
    mesh = plsc.VectorSubcoreMesh(core_axis_name="c", subcore_axis_name="s")

    @functools.partial(
        pl.kernel,
        mesh=mesh,
        compiler_params=pltpu.CompilerParams(
            use_tc_tiling_on_sc=True, needs_layout_passes=False),
        out_type=(
            jax.ShapeDtypeStruct((LATENT, BATCH), jnp.float32),
            jax.ShapeDtypeStruct((LATENT, BATCH), jnp.float32),
            jax.ShapeDtypeStruct((LATENT, BATCH), jnp.float32),
        ),
        scratch_types=[
            pltpu.VMEM((ROWS + 8, 128), jnp.float32),
            pltpu.VMEM((LATENT, ROWS), jnp.float32),
            pltpu.VMEM((LATENT, ROWS), jnp.float32),
            pltpu.VMEM((LATENT, ROWS), jnp.float32),
            pltpu.SemaphoreType.DMA,
        ],
    )
    def k(s1_hbm, s2_hbm, out1, out2, out3, b, o1, o2, o3, sem):
        wid = lax.axis_index("s") * NC + lax.axis_index("c")
        base = wid * ROWS
        lookback = lax.rem(base - 8 + BATCH, BATCH)
        lane = lax.iota(jnp.int32, L)

        pltpu.sync_copy(s1_hbm.at[pl.ds(base, ROWS)], b.at[pl.ds(8, ROWS)])

        def l1_body(ll, _):
            lsplat = lane * 0 + ll
            for j in range(ROWS // L):
                r = lane + j * L
                o1[ll, pl.ds(j * L, L)] = plsc.load_gather(b, [r + 8, lsplat])
            return 0

        lax.fori_loop(0, LATENT, l1_body, 0)
        pltpu.sync_copy(o1, out1.at[:, pl.ds(base, ROWS)])

        pltpu.sync_copy(s2_hbm.at[pl.ds(lookback, 8)], b.at[pl.ds(0, 8)])
        pltpu.sync_copy(s2_hbm.at[pl.ds(base, ROWS)], b.at[pl.ds(8, ROWS)])

        def l2_body(ll, _):
            lsplat = lane * 0 + ll
            for j in range(ROWS // L):
                r = lane + j * L
                o2[ll, pl.ds(j * L, L)] = plsc.load_gather(b, [r + 8, lsplat])
                o3[ll, pl.ds(j * L, L)] = plsc.load_gather(b, [r + 7, lsplat])
            return 0

        lax.fori_loop(0, LATENT, l2_body, 0)
        pltpu.sync_copy(o2, out2.at[:, pl.ds(base, ROWS)])
        pltpu.sync_copy(o3, out3.at[:, pl.ds(base, ROWS)])

    return k(s1, s2)


def kernel(input_ids, table):
    ids_t = input_ids.astype(jnp.int32).T
    table_t = table.T
    tail_t = lax.slice(table_t, (0, TAIL_LO), (LATENT, VOCAB))
    s1, s2 = _phase_a(ids_t, table_t, tail_t)
    o1, o2, o3 = _phase_b(s1, s2)
    return (o1.T, o2.T, o3.T)


# speedup vs baseline: 1.6369x; 1.6369x over previous
import functools

import jax
import jax.numpy as jnp
from jax import lax
from jax.experimental import pallas as pl
from jax.experimental.pallas import tpu as pltpu
from jax.experimental.pallas import tpu_sc as plsc

VOCAB = 1000000
LATENT = 32
BATCH = 16384

NC = 2
NS = 16
NW = NC * NS
L = 16

ROWS = BATCH // NW
VPW = VOCAB // NW
PW = 512
N_PIECES = (VPW + 256 + PW - 1) // PW
MCAP = 4096
NB = 62
BCAP = 64
TRASH = BATCH
SROWS = BATCH + 8
BIGV = 1 << 30
TAIL_LO = (VOCAB // 128) * 128
TAIL_W = VOCAB - TAIL_LO
LAST_PIECE = ((VOCAB - TAIL_W - PW) // 128) * 128


def _gather_rows(ids_flat, table):
    mesh = plsc.VectorSubcoreMesh(core_axis_name="c", subcore_axis_name="s")

    @functools.partial(
        pl.kernel,
        mesh=mesh,
        compiler_params=pltpu.CompilerParams(
            use_tc_tiling_on_sc=False, needs_layout_passes=False),
        out_type=(
            jax.ShapeDtypeStruct((SROWS, 128), jnp.float32),
            jax.ShapeDtypeStruct((SROWS, 128), jnp.float32),
        ),
        scratch_types=[
            pltpu.VMEM((2 * ROWS,), jnp.int32),
            pltpu.VMEM((ROWS // 128, 128), jnp.int32),
            pltpu.VMEM((ROWS // 128, 128), jnp.int32),
            pltpu.VMEM((ROWS, LATENT), jnp.float32),
            pltpu.VMEM((ROWS, LATENT), jnp.float32),
            pltpu.SemaphoreType.DMA,
        ],
    )
    def k(ids_hbm, table_hbm, s1, s2,
          raw_v, idx1_v, idx2_v, rows1_v, rows2_v, sem):
        wid = lax.axis_index("s") * NC + lax.axis_index("c")
        base = wid * ROWS
        lane = lax.iota(jnp.int32, L)

        pltpu.sync_copy(ids_hbm.at[pl.ds(2 * base, 2 * ROWS)], raw_v)

        for t in range(ROWS // L):
            j = 2 * (t * L) + 2 * lane
            v1 = plsc.load_gather(raw_v, [j])
            v2 = plsc.load_gather(raw_v, [j + 1])
            r = (t * L) // 128
            c = (t * L) % 128
            idx1_v[r, pl.ds(c, L)] = v1
            idx2_v[r, pl.ds(c, L)] = v2

        copies = []
        for j in range(ROWS // 128):
            copies.append(pltpu.make_async_copy(
                table_hbm.at[idx1_v.at[j]],
                rows1_v.at[pl.ds(j * 128, 128)], sem))
            copies.append(pltpu.make_async_copy(
                table_hbm.at[idx2_v.at[j]],
                rows2_v.at[pl.ds(j * 128, 128)], sem))
        for cp in copies:
            cp.start()
        for cp in copies:
            cp.wait()

        pltpu.sync_copy(rows1_v, s1.at[pl.ds(base, ROWS), pl.ds(0, LATENT)])
        pltpu.sync_copy(rows2_v, s2.at[pl.ds(base, ROWS), pl.ds(0, LATENT)])

    return k(ids_flat, table)


def _phase_b(s1, s2):
    mesh = plsc.VectorSubcoreMesh(core_axis_name="c", subcore_axis_name="s")

    @functools.partial(
        pl.kernel,
        mesh=mesh,
        compiler_params=pltpu.CompilerParams(
            use_tc_tiling_on_sc=True, needs_layout_passes=False),
        out_type=(
            jax.ShapeDtypeStruct((LATENT, BATCH), jnp.float32),
            jax.ShapeDtypeStruct((LATENT, BATCH), jnp.float32),
            jax.ShapeDtypeStruct((LATENT, BATCH), jnp.float32),
        ),
        scratch_types=[
            pltpu.VMEM((ROWS + 8, 128), jnp.float32),
            pltpu.VMEM((LATENT, ROWS), jnp.float32),
            pltpu.VMEM((LATENT, ROWS), jnp.float32),
            pltpu.VMEM((LATENT, ROWS), jnp.float32),
            pltpu.SemaphoreType.DMA,
        ],
    )
    def k(s1_hbm, s2_hbm, out1, out2, out3, b, o1, o2, o3, sem):
        wid = lax.axis_index("s") * NC + lax.axis_index("c")
        base = wid * ROWS
        lookback = lax.rem(base - 8 + BATCH, BATCH)
        lane = lax.iota(jnp.int32, L)

        pltpu.sync_copy(s1_hbm.at[pl.ds(base, ROWS)], b.at[pl.ds(8, ROWS)])

        def l1_body(ll, _):
            lsplat = lane * 0 + ll
            for j in range(ROWS // L):
                r = lane + j * L
                o1[ll, pl.ds(j * L, L)] = plsc.load_gather(b, [r + 8, lsplat])
            return 0

        lax.fori_loop(0, LATENT, l1_body, 0)
        pltpu.sync_copy(o1, out1.at[:, pl.ds(base, ROWS)])

        pltpu.sync_copy(s2_hbm.at[pl.ds(lookback, 8)], b.at[pl.ds(0, 8)])
        pltpu.sync_copy(s2_hbm.at[pl.ds(base, ROWS)], b.at[pl.ds(8, ROWS)])

        def l2_body(ll, _):
            lsplat = lane * 0 + ll
            for j in range(ROWS // L):
                r = lane + j * L
                o2[ll, pl.ds(j * L, L)] = plsc.load_gather(b, [r + 8, lsplat])
                o3[ll, pl.ds(j * L, L)] = plsc.load_gather(b, [r + 7, lsplat])
            return 0

        lax.fori_loop(0, LATENT, l2_body, 0)
        pltpu.sync_copy(o2, out2.at[:, pl.ds(base, ROWS)])
        pltpu.sync_copy(o3, out3.at[:, pl.ds(base, ROWS)])

    return k(s1, s2)


def kernel(input_ids, table):
    ids_flat = input_ids.astype(jnp.int32).reshape(2 * BATCH)
    s1, s2 = _gather_rows(ids_flat, table)
    o1, o2, o3 = _phase_b(s1, s2)
    return (o1.T, o2.T, o3.T)


# speedup vs baseline: 1.6877x; 1.0310x over previous
import functools

import jax
import jax.numpy as jnp
from jax import lax
from jax.experimental import pallas as pl
from jax.experimental.pallas import tpu as pltpu
from jax.experimental.pallas import tpu_sc as plsc

VOCAB = 1000000
LATENT = 32
BATCH = 16384

NC = 2
NS = 16
NW = NC * NS
L = 16

ROWS = BATCH // NW
KCH = 128
NCH = ROWS // KCH
PAD = 8


def _sc_lookup(ids_flat, table):
    mesh = plsc.VectorSubcoreMesh(core_axis_name="c", subcore_axis_name="s")

    @functools.partial(
        pl.kernel,
        mesh=mesh,
        compiler_params=pltpu.CompilerParams(
            use_tc_tiling_on_sc=False,
            needs_layout_passes=False,
        ),
        out_type=(
            jax.ShapeDtypeStruct((BATCH, LATENT), jnp.float32),
            jax.ShapeDtypeStruct((BATCH, LATENT), jnp.float32),
            jax.ShapeDtypeStruct((BATCH, LATENT), jnp.float32),
        ),
        scratch_types=[
            pltpu.VMEM((2 * ROWS + PAD,), jnp.int32),
            pltpu.VMEM((NCH, KCH), jnp.int32),
            pltpu.VMEM((NCH, KCH), jnp.int32),
            pltpu.VMEM((NCH, KCH), jnp.int32),
            pltpu.VMEM((ROWS, LATENT), jnp.float32),
            pltpu.VMEM((ROWS, LATENT), jnp.float32),
            pltpu.VMEM((ROWS, LATENT), jnp.float32),
            pltpu.SemaphoreType.DMA,
        ],
    )
    def k(ids_hbm, table_hbm, out1, out2, out3,
          raw_v, idx1_v, idx2_v, idx3_v, rows1_v, rows2_v, rows3_v, sem):
        wid = lax.axis_index("s") * NC + lax.axis_index("c")
        base = wid * ROWS

        prev = lax.rem(2 * base - PAD + 2 * BATCH, 2 * BATCH)
        pltpu.sync_copy(ids_hbm.at[pl.ds(prev, PAD)], raw_v.at[pl.ds(0, PAD)])
        pltpu.sync_copy(ids_hbm.at[pl.ds(2 * base, 2 * ROWS)],
                        raw_v.at[pl.ds(PAD, 2 * ROWS)])

        lane = lax.iota(jnp.int32, L)
        for t in range(ROWS // L):
            j = 2 * (t * L) + 2 * lane
            v3 = plsc.load_gather(raw_v, [j + (PAD - 1)])
            v1 = plsc.load_gather(raw_v, [j + PAD])
            v2 = plsc.load_gather(raw_v, [j + (PAD + 1)])
            r = (t * L) // KCH
            c = (t * L) % KCH
            idx1_v[r, pl.ds(c, L)] = v1
            idx2_v[r, pl.ds(c, L)] = v2
            idx3_v[r, pl.ds(c, L)] = v3

        copies = []
        for j in range(NCH):
            copies.append(pltpu.make_async_copy(
                table_hbm.at[idx1_v.at[j]],
                rows1_v.at[pl.ds(j * KCH, KCH)], sem))
            copies.append(pltpu.make_async_copy(
                table_hbm.at[idx2_v.at[j]],
                rows2_v.at[pl.ds(j * KCH, KCH)], sem))
            copies.append(pltpu.make_async_copy(
                table_hbm.at[idx3_v.at[j]],
                rows3_v.at[pl.ds(j * KCH, KCH)], sem))
        for cp in copies:
            cp.start()
        for cp in copies:
            cp.wait()

        pltpu.sync_copy(rows1_v, out1.at[pl.ds(base, ROWS)])
        pltpu.sync_copy(rows2_v, out2.at[pl.ds(base, ROWS)])
        pltpu.sync_copy(rows3_v, out3.at[pl.ds(base, ROWS)])

    return k(ids_flat, table)


def kernel(input_ids, table):
    ids_flat = input_ids.astype(jnp.int32).reshape(2 * BATCH)
    return _sc_lookup(ids_flat, table)
